# baseline (speedup 1.0000x reference)
import dataclasses
import functools

import jax
import jax.numpy as jnp
from jax import lax
from jax.experimental import pallas as pl
from jax.experimental.pallas import tpu as pltpu
from jax.experimental.pallas import tpu_sc as plsc

N = 10000
E = 320000
D = 128
R = 8
NR = N * R

NTILES = 16
S = 12288
P = 4
PTOT = P * 2 * S
RT = S // NTILES
EPT = 20480
EP = NTILES * EPT
LC = 2048
NCH = EPT // LC
PAD_DST = 1 << 20


def _sc_compiler_params():
    cp = pltpu.CompilerParams()
    if "needs_layout_passes" in pltpu.CompilerParams.__dataclass_fields__:
        cp = dataclasses.replace(cp, needs_layout_passes=False)
    return cp


def _sc_accumulate(src_p, dst_p, w_p, x):
    mesh = plsc.VectorSubcoreMesh(core_axis_name="c", subcore_axis_name="s")

    @functools.partial(
        pl.kernel,
        out_type=[
            jax.ShapeDtypeStruct((PTOT, D), jnp.float32),
            jax.ShapeDtypeStruct((PTOT,), jnp.float32),
        ],
        mesh=mesh,
        scratch_types=[
            pltpu.VMEM_SHARED((S, D), jnp.float32),
            pltpu.VMEM_SHARED((S,), jnp.float32),
            pltpu.VMEM((LC,), jnp.int32),
            pltpu.VMEM((LC,), jnp.int32),
            pltpu.VMEM((LC,), jnp.float32),
            pltpu.VMEM((256,), jnp.int32),
            pltpu.VMEM((256,), jnp.int32),
            pltpu.VMEM((256,), jnp.float32),
            pltpu.VMEM((128,), jnp.int32),
            pltpu.VMEM((1, 128), jnp.int32),
            pltpu.VMEM((128,), jnp.float32),
            pltpu.VMEM((128, D), jnp.float32),
            pltpu.VMEM((RT,), jnp.float32),
        ],
        compiler_params=_sc_compiler_params(),
    )
    def sc_kernel(src_hbm, dst_hbm, w_hbm, x_hbm, a_hbm, d_hbm,
                  a_sh, d_sh, src_v, dst_v, w_v,
                  stg_src, stg_row, stg_w, fsrc, frow, fw,
                  rows_v, zero1_v):
        scid = lax.axis_index("c")
        tid = lax.axis_index("s")
        z16 = jnp.zeros((16,), jnp.float32)
        zi16 = jnp.zeros((16,), jnp.int32)
        iot = lax.iota(jnp.int32, 16)

        @pl.loop(0, RT // 16)
        def _(i):
            zero1_v[pl.ds(i * 16, 16)] = z16

        def fire():
            for k in range(8):
                sl = pl.ds(k * 16, 16)
                fsrc[sl] = stg_src[sl]
                frow[0, sl] = stg_row[sl]
                fw[sl] = stg_w[sl]
            pltpu.sync_copy(x_hbm.at[fsrc], rows_v)

            @pl.loop(0, 8)
            def _(eg):
                wg = fw[pl.ds(eg * 16, 16)]
                for l in range(16):
                    e = eg * 16 + l
                    wv = wg[l]
                    for cb in range(D // 16):
                        csl = pl.ds(cb * 16, 16)
                        rows_v[e, csl] = rows_v[e, csl] * wv

            pltpu.sync_copy(rows_v, a_sh.at[frow.at[0]], add=True)
            pltpu.sync_copy(fw, d_sh.at[frow.at[0]], add=True)
            for k in range(8):
                dst_sl = pl.ds(k * 16, 16)
                src_sl = pl.ds(128 + k * 16, 16)
                stg_src[dst_sl] = stg_src[src_sl]
                stg_row[dst_sl] = stg_row[src_sl]
                stg_w[dst_sl] = stg_w[src_sl]

        for p in range(P):
            lo = jnp.int32(p * 2 * S) + scid * S
            hi = lo + S

            @pl.loop(0, 128)
            def _(i):
                for k in range(D // 16):
                    rows_v[i, pl.ds(k * 16, 16)] = z16

            for k in range(RT // 128):
                pltpu.sync_copy(rows_v,
                                a_sh.at[pl.ds(tid * RT + k * 128, 128)])
            pltpu.sync_copy(zero1_v, d_sh.at[pl.ds(tid * RT, RT)])
            plsc.subcore_barrier()

            def batch_body(b, c):
                for g in range(8):
                    off = b * 128 + g * 16
                    d16 = dst_v[pl.ds(off, 16)]
                    s16 = src_v[pl.ds(off, 16)]
                    w16 = w_v[pl.ds(off, 16)]
                    m = jnp.logical_and(d16 >= lo, d16 < hi)
                    plsc.store_compressed(stg_row.at[pl.ds(c, 16)], d16 - lo, mask=m)
                    plsc.store_compressed(stg_src.at[pl.ds(c, 16)], s16, mask=m)
                    plsc.store_compressed(stg_w.at[pl.ds(c, 16)], w16, mask=m)
                    c = c + jnp.sum(m.astype(jnp.int32))

                @pl.when(c >= 128)
                def _():
                    fire()

                return jnp.where(c >= 128, c - 128, c)

            def chunk_body(ch, c):
                eb = tid * EPT + ch * LC
                pltpu.sync_copy(src_hbm.at[pl.ds(eb, LC)], src_v)
                pltpu.sync_copy(dst_hbm.at[pl.ds(eb, LC)], dst_v)
                pltpu.sync_copy(w_hbm.at[pl.ds(eb, LC)], w_v)
                return lax.fori_loop(0, LC // 128, batch_body, c)

            c = lax.fori_loop(0, NCH, chunk_body, jnp.int32(0))

            for k in range(8):
                sl = pl.ds(c + k * 16, 16)
                stg_row[sl] = iot + (k * 16)
                stg_src[sl] = zi16
                stg_w[sl] = z16

            def flush_body(cc):
                fire()
                return cc - 128

            c = lax.while_loop(lambda cc: cc > 0, flush_body, c)
            plsc.subcore_barrier()

            gbase = jnp.int32(p * 2 * S) + scid * S + tid * RT
            pltpu.sync_copy(a_sh.at[pl.ds(tid * RT, RT)],
                            a_hbm.at[pl.ds(gbase, RT)])
            pltpu.sync_copy(d_sh.at[pl.ds(tid * RT, RT)],
                            d_hbm.at[pl.ds(gbase, RT)])
            plsc.subcore_barrier()

    return sc_kernel(src_p, dst_p, w_p, x)


TN = 1000


def _tc_body(a_ref, d_ref, x_ref, wl_ref, ws_ref, bl_ref, bs_ref, o_ref):
    r = pl.program_id(1)
    dall = d_ref[...]
    colmask = lax.broadcasted_iota(jnp.int32, (TN, R), 1) == r
    dcol = jnp.sum(jnp.where(colmask, dall, 0.0), axis=1, keepdims=True)
    dinv = jnp.where(dcol > 0.0, 1.0 / dcol, 0.0)
    u = a_ref[...] * dinv
    acc = lax.dot_general(u, wl_ref[...], (((1,), (1,)), ((), ())),
                          preferred_element_type=jnp.float32)

    @pl.when(r == 0)
    def _():
        o_ref[...] = (
            lax.dot_general(x_ref[...], ws_ref[...],
                            (((1,), (1,)), ((), ())),
                            preferred_element_type=jnp.float32)
            + bl_ref[...] + bs_ref[...]
        )

    o_ref[...] += acc

    @pl.when(r == R - 1)
    def _():
        o_ref[...] = jnp.maximum(o_ref[...], 0.0)


def _tc_combine(a2, d2, x, W_lin, W_self, b_lin, b_self):
    return pl.pallas_call(
        _tc_body,
        out_shape=jax.ShapeDtypeStruct((N, D), jnp.float32),
        grid=(N // TN, R),
        in_specs=[
            pl.BlockSpec((TN, D), lambda i, r: (i, r)),
            pl.BlockSpec((TN, R), lambda i, r: (i, 0)),
            pl.BlockSpec((TN, D), lambda i, r: (i, 0)),
            pl.BlockSpec((D, D), lambda i, r: (0, r)),
            pl.BlockSpec((D, D), lambda i, r: (0, 0)),
            pl.BlockSpec((1, D), lambda i, r: (0, 0)),
            pl.BlockSpec((1, D), lambda i, r: (0, 0)),
        ],
        out_specs=pl.BlockSpec((TN, D), lambda i, r: (i, 0)),
    )(a2, d2, x, W_lin, W_self,
      b_lin.reshape(1, D), b_self.reshape(1, D))


def kernel(x, node_in, node_out, relation, edge_weight, W_lin, b_lin,
           W_self, b_self):
    pad = EP - E
    src_p = jnp.concatenate(
        [node_in.astype(jnp.int32), jnp.zeros((pad,), jnp.int32)])
    dst = node_out.astype(jnp.int32) * R + relation.astype(jnp.int32)
    dst_p = jnp.concatenate([dst, jnp.full((pad,), PAD_DST, jnp.int32)])
    w_p = jnp.concatenate([edge_weight, jnp.zeros((pad,), jnp.float32)])

    a, d = _sc_accumulate(src_p, dst_p, w_p, x)
    a2 = a.reshape(PTOT // R, R * D)
    d2 = d.reshape(PTOT // R, R)
    return _tc_combine(a2, d2, x, W_lin, W_self, b_lin, b_self)


# speedup vs baseline: 1.3232x; 1.3232x over previous
import dataclasses
import functools

import jax
import jax.numpy as jnp
from jax import lax
from jax.experimental import pallas as pl
from jax.experimental.pallas import tpu as pltpu
from jax.experimental.pallas import tpu_sc as plsc

N = 10000
E = 320000
D = 128
R = 8
NR = N * R

NTILES = 16
S = 12288
P = 4
PTOT = P * 2 * S
RT = S // NTILES
EPT = 20480
EP = NTILES * EPT
LC = 2048
NCH = EPT // LC
G = 64
PAD_DST = 1 << 20


def _sc_compiler_params():
    cp = pltpu.CompilerParams()
    if "needs_layout_passes" in pltpu.CompilerParams.__dataclass_fields__:
        cp = dataclasses.replace(cp, needs_layout_passes=False)
    return cp


def _sc_accumulate(src_p, dst_p, w_p, x):
    mesh = plsc.VectorSubcoreMesh(core_axis_name="c", subcore_axis_name="s")

    @functools.partial(
        pl.kernel,
        out_type=[
            jax.ShapeDtypeStruct((PTOT, D), jnp.float32),
            jax.ShapeDtypeStruct((PTOT,), jnp.float32),
        ],
        mesh=mesh,
        scratch_types=[
            pltpu.VMEM_SHARED((S, D), jnp.float32),
            pltpu.VMEM_SHARED((S,), jnp.float32),
            pltpu.VMEM((LC,), jnp.int32),
            pltpu.VMEM((LC,), jnp.int32),
            pltpu.VMEM((LC,), jnp.float32),
            pltpu.VMEM((256,), jnp.int32),
            pltpu.VMEM((256,), jnp.int32),
            pltpu.VMEM((256,), jnp.float32),
            pltpu.VMEM((G,), jnp.int32),
            pltpu.VMEM((G,), jnp.int32),
            pltpu.VMEM((1, G), jnp.int32),
            pltpu.VMEM((1, G), jnp.int32),
            pltpu.VMEM((G,), jnp.float32),
            pltpu.VMEM((G,), jnp.float32),
            pltpu.VMEM((G, D), jnp.float32),
            pltpu.VMEM((G, D), jnp.float32),
            pltpu.VMEM((RT,), jnp.float32),
            pltpu.SemaphoreType.DMA,
            pltpu.SemaphoreType.DMA,
            pltpu.SemaphoreType.DMA,
            pltpu.SemaphoreType.DMA,
            pltpu.SemaphoreType.DMA,
            pltpu.SemaphoreType.DMA,
        ],
        compiler_params=_sc_compiler_params(),
    )
    def sc_kernel(src_hbm, dst_hbm, w_hbm, x_hbm, a_hbm, d_hbm,
                  a_sh, d_sh, src_v, dst_v, w_v,
                  stg_src, stg_row, stg_w,
                  fsrc0, fsrc1, frow0, frow1, fw0, fw1, rows0, rows1,
                  zero1_v, gsem0, gsem1, ssem0, ssem1, dsem0, dsem1):
        scid = lax.axis_index("c")
        tid = lax.axis_index("s")
        z16 = jnp.zeros((16,), jnp.float32)
        zi16 = jnp.zeros((16,), jnp.int32)
        iot = lax.iota(jnp.int32, 16)

        set0 = (fsrc0, frow0, fw0, rows0, gsem0, ssem0, dsem0)
        set1 = (fsrc1, frow1, fw1, rows1, gsem1, ssem1, dsem1)

        @pl.loop(0, RT // 16)
        def _(i):
            zero1_v[pl.ds(i * 16, 16)] = z16

        def wait_scatters(st):
            _, frow_x, fw_x, rows_x, _, ssem_x, dsem_x = st
            pltpu.make_async_copy(rows_x, a_sh.at[frow_x.at[0]], ssem_x).wait()
            pltpu.make_async_copy(fw_x, d_sh.at[frow_x.at[0]], dsem_x).wait()

        def complete_prev(st):
            fsrc_x, frow_x, fw_x, rows_x, gsem_x, ssem_x, dsem_x = st
            pltpu.make_async_copy(x_hbm.at[fsrc_x], rows_x, gsem_x).wait()

            @pl.loop(0, G // 16)
            def _(eg):
                wg = fw_x[pl.ds(eg * 16, 16)]
                for l in range(16):
                    e = eg * 16 + l
                    wv = wg[l]
                    for cb in range(D // 16):
                        csl = pl.ds(cb * 16, 16)
                        rows_x[e, csl] = rows_x[e, csl] * wv

            pltpu.async_copy(rows_x, a_sh.at[frow_x.at[0]], ssem_x, add=True)
            pltpu.async_copy(fw_x, d_sh.at[frow_x.at[0]], dsem_x, add=True)

        def fire_step(cur, prev, nf):
            fsrc_c, frow_c, fw_c, rows_c, gsem_c, _, _ = cur

            @pl.when(nf >= 2)
            def _():
                wait_scatters(cur)

            for k in range(G // 16):
                sl = pl.ds(k * 16, 16)
                fsrc_c[sl] = stg_src[sl]
                frow_c[0, sl] = stg_row[sl]
                fw_c[sl] = stg_w[sl]
            for k in range(8):
                dsl = pl.ds(k * 16, 16)
                ssl = pl.ds(G + k * 16, 16)
                stg_src[dsl] = stg_src[ssl]
                stg_row[dsl] = stg_row[ssl]
                stg_w[dsl] = stg_w[ssl]
            pltpu.async_copy(x_hbm.at[fsrc_c], rows_c, gsem_c)

            @pl.when(nf >= 1)
            def _():
                complete_prev(prev)

        def force_fire(nf):
            @pl.when((nf & 1) == 0)
            def _():
                fire_step(set0, set1, nf)

            @pl.when((nf & 1) == 1)
            def _():
                fire_step(set1, set0, nf)

            return nf + 1

        def maybe_fire(c, nf):
            pred = c >= G

            @pl.when(pred & ((nf & 1) == 0))
            def _():
                fire_step(set0, set1, nf)

            @pl.when(pred & ((nf & 1) == 1))
            def _():
                fire_step(set1, set0, nf)

            return jnp.where(pred, c - G, c), jnp.where(pred, nf + 1, nf)

        @pl.loop(0, P)
        def _(p):
            lo = p * (2 * S) + scid * S
            hi = lo + S

            @pl.loop(0, G)
            def _(i):
                for k in range(D // 16):
                    rows0[i, pl.ds(k * 16, 16)] = z16

            for k in range(RT // G):
                pltpu.sync_copy(rows0,
                                a_sh.at[pl.ds(tid * RT + k * G, G)])
            pltpu.sync_copy(zero1_v, d_sh.at[pl.ds(tid * RT, RT)])
            plsc.subcore_barrier()

            def batch_body(b, carry):
                c, nf = carry
                for g in range(8):
                    off = b * 128 + g * 16
                    d16 = dst_v[pl.ds(off, 16)]
                    s16 = src_v[pl.ds(off, 16)]
                    w16 = w_v[pl.ds(off, 16)]
                    m = jnp.logical_and(d16 >= lo, d16 < hi)
                    plsc.store_compressed(stg_row.at[pl.ds(c, 16)],
                                          d16 - lo, mask=m)
                    plsc.store_compressed(stg_src.at[pl.ds(c, 16)],
                                          s16, mask=m)
                    plsc.store_compressed(stg_w.at[pl.ds(c, 16)],
                                          w16, mask=m)
                    c = c + jnp.sum(m.astype(jnp.int32))
                c, nf = maybe_fire(c, nf)
                c, nf = maybe_fire(c, nf)
                return (c, nf)

            def chunk_body(ch, carry):
                eb = tid * EPT + ch * LC
                pltpu.sync_copy(src_hbm.at[pl.ds(eb, LC)], src_v)
                pltpu.sync_copy(dst_hbm.at[pl.ds(eb, LC)], dst_v)
                pltpu.sync_copy(w_hbm.at[pl.ds(eb, LC)], w_v)
                return lax.fori_loop(0, LC // 128, batch_body, carry)

            c, nf = lax.fori_loop(0, NCH, chunk_body,
                                  (jnp.int32(0), jnp.int32(0)))

            for k in range(G // 16):
                sl = pl.ds(c + k * 16, 16)
                stg_row[sl] = iot + (k * 16)
                stg_src[sl] = zi16
                stg_w[sl] = z16

            def flush_body(carry):
                cc, nf = carry
                nf = force_fire(nf)
                return (cc - G, nf)

            c, nf = lax.while_loop(lambda cn: cn[0] > 0, flush_body, (c, nf))

            kf = nf

            @pl.when((kf >= 1) & (((kf - 1) & 1) == 0))
            def _():
                complete_prev(set0)

            @pl.when((kf >= 1) & (((kf - 1) & 1) == 1))
            def _():
                complete_prev(set1)

            @pl.when((kf >= 2) & ((kf & 1) == 0))
            def _():
                wait_scatters(set0)

            @pl.when((kf >= 2) & ((kf & 1) == 1))
            def _():
                wait_scatters(set1)

            @pl.when((kf >= 1) & (((kf - 1) & 1) == 0))
            def _():
                wait_scatters(set0)

            @pl.when((kf >= 1) & (((kf - 1) & 1) == 1))
            def _():
                wait_scatters(set1)

            plsc.subcore_barrier()

            gbase = p * (2 * S) + scid * S + tid * RT
            pltpu.sync_copy(a_sh.at[pl.ds(tid * RT, RT)],
                            a_hbm.at[pl.ds(gbase, RT)])
            pltpu.sync_copy(d_sh.at[pl.ds(tid * RT, RT)],
                            d_hbm.at[pl.ds(gbase, RT)])
            plsc.subcore_barrier()

    return sc_kernel(src_p, dst_p, w_p, x)


TN = 1000


def _tc_body(a_ref, d_ref, x_ref, wl_ref, ws_ref, bl_ref, bs_ref, o_ref):
    r = pl.program_id(1)
    dall = d_ref[...]
    colmask = lax.broadcasted_iota(jnp.int32, (TN, R), 1) == r
    dcol = jnp.sum(jnp.where(colmask, dall, 0.0), axis=1, keepdims=True)
    dinv = jnp.where(dcol > 0.0, 1.0 / dcol, 0.0)
    u = a_ref[...] * dinv
    acc = lax.dot_general(u, wl_ref[...], (((1,), (1,)), ((), ())),
                          preferred_element_type=jnp.float32)

    @pl.when(r == 0)
    def _():
        o_ref[...] = (
            lax.dot_general(x_ref[...], ws_ref[...],
                            (((1,), (1,)), ((), ())),
                            preferred_element_type=jnp.float32)
            + bl_ref[...] + bs_ref[...]
        )

    o_ref[...] += acc

    @pl.when(r == R - 1)
    def _():
        o_ref[...] = jnp.maximum(o_ref[...], 0.0)


def _tc_combine(a2, d2, x, W_lin, W_self, b_lin, b_self):
    return pl.pallas_call(
        _tc_body,
        out_shape=jax.ShapeDtypeStruct((N, D), jnp.float32),
        grid=(N // TN, R),
        in_specs=[
            pl.BlockSpec((TN, D), lambda i, r: (i, r)),
            pl.BlockSpec((TN, R), lambda i, r: (i, 0)),
            pl.BlockSpec((TN, D), lambda i, r: (i, 0)),
            pl.BlockSpec((D, D), lambda i, r: (0, r)),
            pl.BlockSpec((D, D), lambda i, r: (0, 0)),
            pl.BlockSpec((1, D), lambda i, r: (0, 0)),
            pl.BlockSpec((1, D), lambda i, r: (0, 0)),
        ],
        out_specs=pl.BlockSpec((TN, D), lambda i, r: (i, 0)),
    )(a2, d2, x, W_lin, W_self,
      b_lin.reshape(1, D), b_self.reshape(1, D))


def kernel(x, node_in, node_out, relation, edge_weight, W_lin, b_lin,
           W_self, b_self):
    pad = EP - E
    src_p = jnp.concatenate(
        [node_in.astype(jnp.int32), jnp.zeros((pad,), jnp.int32)])
    dst = node_out.astype(jnp.int32) * R + relation.astype(jnp.int32)
    dst_p = jnp.concatenate([dst, jnp.full((pad,), PAD_DST, jnp.int32)])
    w_p = jnp.concatenate([edge_weight, jnp.zeros((pad,), jnp.float32)])

    a, d = _sc_accumulate(src_p, dst_p, w_p, x)
    a2 = a.reshape(PTOT // R, R * D)
    d2 = d.reshape(PTOT // R, R)
    return _tc_combine(a2, d2, x, W_lin, W_self, b_lin, b_self)


# speedup vs baseline: 1.4203x; 1.0734x over previous
import dataclasses
import functools

import jax
import jax.numpy as jnp
from jax import lax
from jax.experimental import pallas as pl
from jax.experimental.pallas import tpu as pltpu
from jax.experimental.pallas import tpu_sc as plsc

N = 10000
E = 320000
D = 128
R = 8
NR = N * R

NTILES = 16
S = 12288
P = 4
PTOT = P * 2 * S
RT = S // NTILES
EPT = 20480
EP = NTILES * EPT
LC = 2048
NCH = EPT // LC
G = 64
PAD_DST = 1 << 20


def _sc_compiler_params():
    cp = pltpu.CompilerParams()
    if "needs_layout_passes" in pltpu.CompilerParams.__dataclass_fields__:
        cp = dataclasses.replace(cp, needs_layout_passes=False)
    return cp


def _sc_accumulate(src_p, dst_p, w_p, x):
    mesh = plsc.VectorSubcoreMesh(core_axis_name="c", subcore_axis_name="s")

    @functools.partial(
        pl.kernel,
        out_type=[
            jax.ShapeDtypeStruct((PTOT, D), jnp.float32),
            jax.ShapeDtypeStruct((PTOT,), jnp.float32),
        ],
        mesh=mesh,
        scratch_types=[
            pltpu.VMEM_SHARED((S, D), jnp.float32),
            pltpu.VMEM_SHARED((S,), jnp.float32),
            pltpu.VMEM((LC,), jnp.int32),
            pltpu.VMEM((LC,), jnp.int32),
            pltpu.VMEM((LC,), jnp.float32),
            pltpu.VMEM((256,), jnp.int32),
            pltpu.VMEM((256,), jnp.int32),
            pltpu.VMEM((256,), jnp.float32),
            pltpu.VMEM((G,), jnp.int32),
            pltpu.VMEM((G,), jnp.int32),
            pltpu.VMEM((1, G), jnp.int32),
            pltpu.VMEM((1, G), jnp.int32),
            pltpu.VMEM((G,), jnp.float32),
            pltpu.VMEM((G,), jnp.float32),
            pltpu.VMEM((G, D), jnp.float32),
            pltpu.VMEM((G, D), jnp.float32),
            pltpu.VMEM((RT,), jnp.float32),
            pltpu.SemaphoreType.DMA,
            pltpu.SemaphoreType.DMA,
            pltpu.SemaphoreType.DMA,
            pltpu.SemaphoreType.DMA,
            pltpu.SemaphoreType.DMA,
            pltpu.SemaphoreType.DMA,
            pltpu.SemaphoreType.DMA,
            pltpu.SemaphoreType.DMA,
            pltpu.SemaphoreType.DMA,
        ],
        compiler_params=_sc_compiler_params(),
    )
    def sc_kernel(src_hbm, dst_hbm, w_hbm, x_hbm, a_hbm, d_hbm,
                  a_sh, d_sh, src_v, dst_v, w_v,
                  stg_src, stg_row, stg_w,
                  fsrc0, fsrc1, frow0, frow1, fw0, fw1, rows0, rows1,
                  zero1_v, gsem0, gsem1, ssem0, ssem1, dsem0, dsem1,
                  lsem0, lsem1, lsem2):
        scid = lax.axis_index("c")
        tid = lax.axis_index("s")
        z16 = jnp.zeros((16,), jnp.float32)
        zi16 = jnp.zeros((16,), jnp.int32)
        iot = lax.iota(jnp.int32, 16)

        set0 = (fsrc0, frow0, fw0, rows0, gsem0, ssem0, dsem0)
        set1 = (fsrc1, frow1, fw1, rows1, gsem1, ssem1, dsem1)

        @pl.loop(0, RT // 16)
        def _(i):
            zero1_v[pl.ds(i * 16, 16)] = z16

        def wait_scatters(st):
            _, frow_x, fw_x, rows_x, _, ssem_x, dsem_x = st
            pltpu.make_async_copy(rows_x, a_sh.at[frow_x.at[0]], ssem_x).wait()
            pltpu.make_async_copy(fw_x, d_sh.at[frow_x.at[0]], dsem_x).wait()

        def complete_prev(st):
            fsrc_x, frow_x, fw_x, rows_x, gsem_x, ssem_x, dsem_x = st
            pltpu.make_async_copy(x_hbm.at[fsrc_x], rows_x, gsem_x).wait()

            @pl.loop(0, G // 16)
            def _(eg):
                wg = fw_x[pl.ds(eg * 16, 16)]
                for l in range(16):
                    e = eg * 16 + l
                    wv = wg[l]
                    for cb in range(D // 16):
                        csl = pl.ds(cb * 16, 16)
                        rows_x[e, csl] = rows_x[e, csl] * wv

            pltpu.async_copy(rows_x, a_sh.at[frow_x.at[0]], ssem_x, add=True)
            pltpu.async_copy(fw_x, d_sh.at[frow_x.at[0]], dsem_x, add=True)

        def fire_step(cur, prev, nf):
            fsrc_c, frow_c, fw_c, rows_c, gsem_c, _, _ = cur

            @pl.when(nf >= 2)
            def _():
                wait_scatters(cur)

            for k in range(G // 16):
                sl = pl.ds(k * 16, 16)
                fsrc_c[sl] = stg_src[sl]
                frow_c[0, sl] = stg_row[sl]
                fw_c[sl] = stg_w[sl]
            for k in range(8):
                dsl = pl.ds(k * 16, 16)
                ssl = pl.ds(G + k * 16, 16)
                stg_src[dsl] = stg_src[ssl]
                stg_row[dsl] = stg_row[ssl]
                stg_w[dsl] = stg_w[ssl]
            pltpu.async_copy(x_hbm.at[fsrc_c], rows_c, gsem_c)

            @pl.when(nf >= 1)
            def _():
                complete_prev(prev)

        def force_fire(nf):
            @pl.when((nf & 1) == 0)
            def _():
                fire_step(set0, set1, nf)

            @pl.when((nf & 1) == 1)
            def _():
                fire_step(set1, set0, nf)

            return nf + 1

        def maybe_fire(c, nf):
            pred = c >= G

            @pl.when(pred & ((nf & 1) == 0))
            def _():
                fire_step(set0, set1, nf)

            @pl.when(pred & ((nf & 1) == 1))
            def _():
                fire_step(set1, set0, nf)

            return jnp.where(pred, c - G, c), jnp.where(pred, nf + 1, nf)

        @pl.loop(0, P)
        def _(p):
            lo = p * (2 * S) + scid * S
            hi = lo + S

            @pl.loop(0, G)
            def _(i):
                for k in range(D // 16):
                    rows0[i, pl.ds(k * 16, 16)] = z16

            for k in range(RT // G):
                pltpu.sync_copy(rows0,
                                a_sh.at[pl.ds(tid * RT + k * G, G)])
            pltpu.sync_copy(zero1_v, d_sh.at[pl.ds(tid * RT, RT)])
            plsc.subcore_barrier()

            def batch_body(b, carry):
                c, nf = carry
                for g in range(8):
                    off = b * 128 + g * 16
                    d16 = dst_v[pl.ds(off, 16)]
                    s16 = src_v[pl.ds(off, 16)]
                    w16 = w_v[pl.ds(off, 16)]
                    m = jnp.logical_and(d16 >= lo, d16 < hi)
                    plsc.store_compressed(stg_row.at[pl.ds(c, 16)],
                                          d16 - lo, mask=m)
                    plsc.store_compressed(stg_src.at[pl.ds(c, 16)],
                                          s16, mask=m)
                    plsc.store_compressed(stg_w.at[pl.ds(c, 16)],
                                          w16, mask=m)
                    c = c + jnp.sum(m.astype(jnp.int32))
                c, nf = maybe_fire(c, nf)
                c, nf = maybe_fire(c, nf)
                return (c, nf)

            def chunk_body(ch, carry):
                eb = tid * EPT + ch * LC
                cp0 = pltpu.async_copy(src_hbm.at[pl.ds(eb, LC)], src_v, lsem0)
                cp1 = pltpu.async_copy(dst_hbm.at[pl.ds(eb, LC)], dst_v, lsem1)
                cp2 = pltpu.async_copy(w_hbm.at[pl.ds(eb, LC)], w_v, lsem2)
                cp0.wait()
                cp1.wait()
                cp2.wait()
                return lax.fori_loop(0, LC // 128, batch_body, carry)

            c, nf = lax.fori_loop(0, NCH, chunk_body,
                                  (jnp.int32(0), jnp.int32(0)))

            for k in range(G // 16):
                sl = pl.ds(c + k * 16, 16)
                stg_row[sl] = iot + (k * 16)
                stg_src[sl] = zi16
                stg_w[sl] = z16

            def flush_body(carry):
                cc, nf = carry
                nf = force_fire(nf)
                return (cc - G, nf)

            c, nf = lax.while_loop(lambda cn: cn[0] > 0, flush_body, (c, nf))

            kf = nf

            @pl.when((kf >= 1) & (((kf - 1) & 1) == 0))
            def _():
                complete_prev(set0)

            @pl.when((kf >= 1) & (((kf - 1) & 1) == 1))
            def _():
                complete_prev(set1)

            @pl.when((kf >= 2) & ((kf & 1) == 0))
            def _():
                wait_scatters(set0)

            @pl.when((kf >= 2) & ((kf & 1) == 1))
            def _():
                wait_scatters(set1)

            @pl.when((kf >= 1) & (((kf - 1) & 1) == 0))
            def _():
                wait_scatters(set0)

            @pl.when((kf >= 1) & (((kf - 1) & 1) == 1))
            def _():
                wait_scatters(set1)

            plsc.subcore_barrier()

            gbase = p * (2 * S) + scid * S + tid * RT
            pltpu.sync_copy(a_sh.at[pl.ds(tid * RT, RT)],
                            a_hbm.at[pl.ds(gbase, RT)])
            pltpu.sync_copy(d_sh.at[pl.ds(tid * RT, RT)],
                            d_hbm.at[pl.ds(gbase, RT)])
            plsc.subcore_barrier()

    return sc_kernel(src_p, dst_p, w_p, x)


TN = 1000


def _tc_body(a_ref, d_ref, x_ref, wl_ref, ws_ref, bl_ref, bs_ref, o_ref):
    r = pl.program_id(1)
    dall = d_ref[...]
    colmask = lax.broadcasted_iota(jnp.int32, (TN, R), 1) == r
    dcol = jnp.sum(jnp.where(colmask, dall, 0.0), axis=1, keepdims=True)
    dinv = jnp.where(dcol > 0.0, 1.0 / dcol, 0.0)
    u = a_ref[...] * dinv
    acc = lax.dot_general(u, wl_ref[...], (((1,), (1,)), ((), ())),
                          preferred_element_type=jnp.float32)

    @pl.when(r == 0)
    def _():
        o_ref[...] = (
            lax.dot_general(x_ref[...], ws_ref[...],
                            (((1,), (1,)), ((), ())),
                            preferred_element_type=jnp.float32)
            + bl_ref[...] + bs_ref[...]
        )

    o_ref[...] += acc

    @pl.when(r == R - 1)
    def _():
        o_ref[...] = jnp.maximum(o_ref[...], 0.0)


def _tc_combine(a2, d2, x, W_lin, W_self, b_lin, b_self):
    return pl.pallas_call(
        _tc_body,
        out_shape=jax.ShapeDtypeStruct((N, D), jnp.float32),
        grid=(N // TN, R),
        in_specs=[
            pl.BlockSpec((TN, D), lambda i, r: (i, r)),
            pl.BlockSpec((TN, R), lambda i, r: (i, 0)),
            pl.BlockSpec((TN, D), lambda i, r: (i, 0)),
            pl.BlockSpec((D, D), lambda i, r: (0, r)),
            pl.BlockSpec((D, D), lambda i, r: (0, 0)),
            pl.BlockSpec((1, D), lambda i, r: (0, 0)),
            pl.BlockSpec((1, D), lambda i, r: (0, 0)),
        ],
        out_specs=pl.BlockSpec((TN, D), lambda i, r: (i, 0)),
    )(a2, d2, x, W_lin, W_self,
      b_lin.reshape(1, D), b_self.reshape(1, D))


def kernel(x, node_in, node_out, relation, edge_weight, W_lin, b_lin,
           W_self, b_self):
    pad = EP - E
    src_p = jnp.concatenate(
        [node_in.astype(jnp.int32), jnp.zeros((pad,), jnp.int32)])
    dst = node_out.astype(jnp.int32) * R + relation.astype(jnp.int32)
    dst_p = jnp.concatenate([dst, jnp.full((pad,), PAD_DST, jnp.int32)])
    w_p = jnp.concatenate([edge_weight, jnp.zeros((pad,), jnp.float32)])

    a, d = _sc_accumulate(src_p, dst_p, w_p, x)
    a2 = a.reshape(PTOT // R, R * D)
    d2 = d.reshape(PTOT // R, R)
    return _tc_combine(a2, d2, x, W_lin, W_self, b_lin, b_self)


# speedup vs baseline: 1.4389x; 1.0131x over previous
import dataclasses
import functools

import jax
import jax.numpy as jnp
from jax import lax
from jax.experimental import pallas as pl
from jax.experimental.pallas import tpu as pltpu
from jax.experimental.pallas import tpu_sc as plsc

N = 10000
E = 320000
D = 128
R = 8
NR = N * R

NTILES = 16
S = 12288
P = 4
PTOT = P * 2 * S
RT = S // NTILES
EPT = 20480
EP = NTILES * EPT
LC = 2048
NCH = EPT // LC
G = 64
PAD_DST = 1 << 20


def _sc_compiler_params():
    cp = pltpu.CompilerParams()
    if "needs_layout_passes" in pltpu.CompilerParams.__dataclass_fields__:
        cp = dataclasses.replace(cp, needs_layout_passes=False)
    return cp


def _sc_accumulate(src_p, dst_p, w_p, x):
    mesh = plsc.VectorSubcoreMesh(core_axis_name="c", subcore_axis_name="s")

    @functools.partial(
        pl.kernel,
        out_type=[
            jax.ShapeDtypeStruct((PTOT, D), jnp.float32),
            jax.ShapeDtypeStruct((PTOT,), jnp.float32),
        ],
        mesh=mesh,
        scratch_types=[
            pltpu.VMEM_SHARED((S, D), jnp.float32),
            pltpu.VMEM_SHARED((S,), jnp.float32),
            pltpu.VMEM((LC,), jnp.int32),
            pltpu.VMEM((LC,), jnp.int32),
            pltpu.VMEM((LC,), jnp.float32),
            pltpu.VMEM((256,), jnp.int32),
            pltpu.VMEM((256,), jnp.int32),
            pltpu.VMEM((256,), jnp.float32),
            pltpu.VMEM((G,), jnp.int32),
            pltpu.VMEM((G,), jnp.int32),
            pltpu.VMEM((1, G), jnp.int32),
            pltpu.VMEM((1, G), jnp.int32),
            pltpu.VMEM((G,), jnp.float32),
            pltpu.VMEM((G,), jnp.float32),
            pltpu.VMEM((G, D), jnp.float32),
            pltpu.VMEM((G, D), jnp.float32),
            pltpu.VMEM((RT,), jnp.float32),
            pltpu.SemaphoreType.DMA,
            pltpu.SemaphoreType.DMA,
            pltpu.SemaphoreType.DMA,
            pltpu.SemaphoreType.DMA,
            pltpu.SemaphoreType.DMA,
            pltpu.SemaphoreType.DMA,
            pltpu.SemaphoreType.DMA,
            pltpu.SemaphoreType.DMA,
            pltpu.SemaphoreType.DMA,
        ],
        compiler_params=_sc_compiler_params(),
    )
    def sc_kernel(src_hbm, dst_hbm, w_hbm, x_hbm, a_hbm, d_hbm,
                  a_sh, d_sh, src_v, dst_v, w_v,
                  stg_src, stg_row, stg_w,
                  fsrc0, fsrc1, frow0, frow1, fw0, fw1, rows0, rows1,
                  zero1_v, gsem0, gsem1, ssem0, ssem1, dsem0, dsem1,
                  lsem0, lsem1, lsem2):
        scid = lax.axis_index("c")
        tid = lax.axis_index("s")
        z16 = jnp.zeros((16,), jnp.float32)
        zi16 = jnp.zeros((16,), jnp.int32)
        iot = lax.iota(jnp.int32, 16)

        set0 = (fsrc0, frow0, fw0, rows0, gsem0, ssem0, dsem0)
        set1 = (fsrc1, frow1, fw1, rows1, gsem1, ssem1, dsem1)

        @pl.loop(0, RT // 16)
        def _(i):
            zero1_v[pl.ds(i * 16, 16)] = z16

        def wait_scatters(st):
            _, frow_x, fw_x, rows_x, _, ssem_x, dsem_x = st
            pltpu.make_async_copy(rows_x, a_sh.at[frow_x.at[0]], ssem_x).wait()
            pltpu.make_async_copy(fw_x, d_sh.at[frow_x.at[0]], dsem_x).wait()

        def complete_prev(st):
            fsrc_x, frow_x, fw_x, rows_x, gsem_x, ssem_x, dsem_x = st
            pltpu.make_async_copy(x_hbm.at[fsrc_x], rows_x, gsem_x).wait()

            @pl.loop(0, G // 16)
            def _(eg):
                wg = fw_x[pl.ds(eg * 16, 16)]
                for l in range(16):
                    e = eg * 16 + l
                    wv = wg[l]
                    for cb in range(D // 16):
                        csl = pl.ds(cb * 16, 16)
                        rows_x[e, csl] = rows_x[e, csl] * wv

            pltpu.async_copy(rows_x, a_sh.at[frow_x.at[0]], ssem_x, add=True)
            pltpu.async_copy(fw_x, d_sh.at[frow_x.at[0]], dsem_x, add=True)

        def fire_step(cur, prev, nf):
            fsrc_c, frow_c, fw_c, rows_c, gsem_c, _, _ = cur

            @pl.when(nf >= 2)
            def _():
                wait_scatters(cur)

            for k in range(G // 16):
                sl = pl.ds(k * 16, 16)
                fsrc_c[sl] = stg_src[sl]
                frow_c[0, sl] = stg_row[sl]
                fw_c[sl] = stg_w[sl]
            for k in range(8):
                dsl = pl.ds(k * 16, 16)
                ssl = pl.ds(G + k * 16, 16)
                stg_src[dsl] = stg_src[ssl]
                stg_row[dsl] = stg_row[ssl]
                stg_w[dsl] = stg_w[ssl]
            pltpu.async_copy(x_hbm.at[fsrc_c], rows_c, gsem_c)

            @pl.when(nf >= 1)
            def _():
                complete_prev(prev)

        def force_fire(nf):
            @pl.when((nf & 1) == 0)
            def _():
                fire_step(set0, set1, nf)

            @pl.when((nf & 1) == 1)
            def _():
                fire_step(set1, set0, nf)

            return nf + 1

        def maybe_fire(c, nf):
            pred = c >= G

            @pl.when(pred & ((nf & 1) == 0))
            def _():
                fire_step(set0, set1, nf)

            @pl.when(pred & ((nf & 1) == 1))
            def _():
                fire_step(set1, set0, nf)

            return jnp.where(pred, c - G, c), jnp.where(pred, nf + 1, nf)

        @pl.loop(0, P)
        def _(p):
            lo = p * (2 * S) + scid * S
            hi = lo + S

            @pl.loop(0, G)
            def _(i):
                for k in range(D // 16):
                    rows0[i, pl.ds(k * 16, 16)] = z16

            for k in range(RT // G):
                pltpu.sync_copy(rows0,
                                a_sh.at[pl.ds(tid * RT + k * G, G)])
            pltpu.sync_copy(zero1_v, d_sh.at[pl.ds(tid * RT, RT)])
            plsc.subcore_barrier()

            def batch_body(b, carry):
                c, nf = carry
                for g in range(8):
                    off = b * 128 + g * 16
                    d16 = dst_v[pl.ds(off, 16)]
                    s16 = src_v[pl.ds(off, 16)]
                    w16 = w_v[pl.ds(off, 16)]
                    m = jnp.logical_and(d16 >= lo, d16 < hi)
                    plsc.store_compressed(stg_row.at[pl.ds(c, 16)],
                                          d16 - lo, mask=m)
                    plsc.store_compressed(stg_src.at[pl.ds(c, 16)],
                                          s16, mask=m)
                    plsc.store_compressed(stg_w.at[pl.ds(c, 16)],
                                          w16, mask=m)
                    cnt = plsc.all_reduce_population_count(m)
                    c = c + cnt[0]
                c, nf = maybe_fire(c, nf)
                c, nf = maybe_fire(c, nf)
                return (c, nf)

            def chunk_body(ch, carry):
                eb = tid * EPT + ch * LC
                cp0 = pltpu.async_copy(src_hbm.at[pl.ds(eb, LC)], src_v, lsem0)
                cp1 = pltpu.async_copy(dst_hbm.at[pl.ds(eb, LC)], dst_v, lsem1)
                cp2 = pltpu.async_copy(w_hbm.at[pl.ds(eb, LC)], w_v, lsem2)
                cp0.wait()
                cp1.wait()
                cp2.wait()
                return lax.fori_loop(0, LC // 128, batch_body, carry)

            c, nf = lax.fori_loop(0, NCH, chunk_body,
                                  (jnp.int32(0), jnp.int32(0)))

            for k in range(G // 16):
                sl = pl.ds(c + k * 16, 16)
                stg_row[sl] = iot + (k * 16)
                stg_src[sl] = zi16
                stg_w[sl] = z16

            def flush_body(carry):
                cc, nf = carry
                nf = force_fire(nf)
                return (cc - G, nf)

            c, nf = lax.while_loop(lambda cn: cn[0] > 0, flush_body, (c, nf))

            kf = nf

            @pl.when((kf >= 1) & (((kf - 1) & 1) == 0))
            def _():
                complete_prev(set0)

            @pl.when((kf >= 1) & (((kf - 1) & 1) == 1))
            def _():
                complete_prev(set1)

            @pl.when((kf >= 2) & ((kf & 1) == 0))
            def _():
                wait_scatters(set0)

            @pl.when((kf >= 2) & ((kf & 1) == 1))
            def _():
                wait_scatters(set1)

            @pl.when((kf >= 1) & (((kf - 1) & 1) == 0))
            def _():
                wait_scatters(set0)

            @pl.when((kf >= 1) & (((kf - 1) & 1) == 1))
            def _():
                wait_scatters(set1)

            plsc.subcore_barrier()

            gbase = p * (2 * S) + scid * S + tid * RT
            pltpu.sync_copy(a_sh.at[pl.ds(tid * RT, RT)],
                            a_hbm.at[pl.ds(gbase, RT)])
            pltpu.sync_copy(d_sh.at[pl.ds(tid * RT, RT)],
                            d_hbm.at[pl.ds(gbase, RT)])
            plsc.subcore_barrier()

    return sc_kernel(src_p, dst_p, w_p, x)


TN = 1000


def _tc_body(a_ref, d_ref, x_ref, wl_ref, ws_ref, bl_ref, bs_ref, o_ref):
    r = pl.program_id(1)
    dall = d_ref[...]
    colmask = lax.broadcasted_iota(jnp.int32, (TN, R), 1) == r
    dcol = jnp.sum(jnp.where(colmask, dall, 0.0), axis=1, keepdims=True)
    dinv = jnp.where(dcol > 0.0, 1.0 / dcol, 0.0)
    u = a_ref[...] * dinv
    acc = lax.dot_general(u, wl_ref[...], (((1,), (1,)), ((), ())),
                          preferred_element_type=jnp.float32)

    @pl.when(r == 0)
    def _():
        o_ref[...] = (
            lax.dot_general(x_ref[...], ws_ref[...],
                            (((1,), (1,)), ((), ())),
                            preferred_element_type=jnp.float32)
            + bl_ref[...] + bs_ref[...]
        )

    o_ref[...] += acc

    @pl.when(r == R - 1)
    def _():
        o_ref[...] = jnp.maximum(o_ref[...], 0.0)


def _tc_combine(a2, d2, x, W_lin, W_self, b_lin, b_self):
    return pl.pallas_call(
        _tc_body,
        out_shape=jax.ShapeDtypeStruct((N, D), jnp.float32),
        grid=(N // TN, R),
        in_specs=[
            pl.BlockSpec((TN, D), lambda i, r: (i, r)),
            pl.BlockSpec((TN, R), lambda i, r: (i, 0)),
            pl.BlockSpec((TN, D), lambda i, r: (i, 0)),
            pl.BlockSpec((D, D), lambda i, r: (0, r)),
            pl.BlockSpec((D, D), lambda i, r: (0, 0)),
            pl.BlockSpec((1, D), lambda i, r: (0, 0)),
            pl.BlockSpec((1, D), lambda i, r: (0, 0)),
        ],
        out_specs=pl.BlockSpec((TN, D), lambda i, r: (i, 0)),
    )(a2, d2, x, W_lin, W_self,
      b_lin.reshape(1, D), b_self.reshape(1, D))


def kernel(x, node_in, node_out, relation, edge_weight, W_lin, b_lin,
           W_self, b_self):
    pad = EP - E
    src_p = jnp.concatenate(
        [node_in.astype(jnp.int32), jnp.zeros((pad,), jnp.int32)])
    dst = node_out.astype(jnp.int32) * R + relation.astype(jnp.int32)
    dst_p = jnp.concatenate([dst, jnp.full((pad,), PAD_DST, jnp.int32)])
    w_p = jnp.concatenate([edge_weight, jnp.zeros((pad,), jnp.float32)])

    a, d = _sc_accumulate(src_p, dst_p, w_p, x)
    a2 = a.reshape(PTOT // R, R * D)
    d2 = d.reshape(PTOT // R, R)
    return _tc_combine(a2, d2, x, W_lin, W_self, b_lin, b_self)


# speedup vs baseline: 1.4461x; 1.0050x over previous
import dataclasses
import functools

import jax
import jax.numpy as jnp
from jax import lax
from jax.experimental import pallas as pl
from jax.experimental.pallas import tpu as pltpu
from jax.experimental.pallas import tpu_sc as plsc

N = 10000
E = 320000
D = 128
R = 8
NR = N * R

NTILES = 16
S = 12288
P = 4
PTOT = P * 2 * S
RT = S // NTILES
EPT = 20480
EP = NTILES * EPT
LC = 2048
NCH = EPT // LC
G = 64
PAD_DST = 1 << 20


def _sc_compiler_params():
    cp = pltpu.CompilerParams()
    if "needs_layout_passes" in pltpu.CompilerParams.__dataclass_fields__:
        cp = dataclasses.replace(cp, needs_layout_passes=False)
    return cp


def _sc_accumulate(src_p, dst_p, w_p, x):
    mesh = plsc.VectorSubcoreMesh(core_axis_name="c", subcore_axis_name="s")

    @functools.partial(
        pl.kernel,
        out_type=[
            jax.ShapeDtypeStruct((PTOT, D), jnp.float32),
            jax.ShapeDtypeStruct((PTOT,), jnp.float32),
        ],
        mesh=mesh,
        scratch_types=[
            pltpu.VMEM_SHARED((S, D), jnp.float32),
            pltpu.VMEM_SHARED((S,), jnp.float32),
            pltpu.VMEM((LC,), jnp.int32),
            pltpu.VMEM((LC,), jnp.int32),
            pltpu.VMEM((LC,), jnp.float32),
            pltpu.VMEM((256,), jnp.int32),
            pltpu.VMEM((256,), jnp.int32),
            pltpu.VMEM((256,), jnp.float32),
            pltpu.VMEM((G,), jnp.int32),
            pltpu.VMEM((G,), jnp.int32),
            pltpu.VMEM((1, G), jnp.int32),
            pltpu.VMEM((1, G), jnp.int32),
            pltpu.VMEM((G,), jnp.float32),
            pltpu.VMEM((G,), jnp.float32),
            pltpu.VMEM((G, D), jnp.float32),
            pltpu.VMEM((G, D), jnp.float32),
            pltpu.VMEM((RT,), jnp.float32),
            pltpu.SemaphoreType.DMA,
            pltpu.SemaphoreType.DMA,
            pltpu.SemaphoreType.DMA,
            pltpu.SemaphoreType.DMA,
            pltpu.SemaphoreType.DMA,
            pltpu.SemaphoreType.DMA,
            pltpu.SemaphoreType.DMA,
            pltpu.SemaphoreType.DMA,
            pltpu.SemaphoreType.DMA,
        ],
        compiler_params=_sc_compiler_params(),
    )
    def sc_kernel(src_hbm, dst_hbm, w_hbm, x_hbm, a_hbm, d_hbm,
                  a_sh, d_sh, src_v, dst_v, w_v,
                  stg_src, stg_row, stg_w,
                  fsrc0, fsrc1, frow0, frow1, fw0, fw1, rows0, rows1,
                  zero1_v, gsem0, gsem1, ssem0, ssem1, dsem0, dsem1,
                  lsem0, lsem1, lsem2):
        scid = lax.axis_index("c")
        tid = lax.axis_index("s")
        z16 = jnp.zeros((16,), jnp.float32)
        zi16 = jnp.zeros((16,), jnp.int32)
        iot = lax.iota(jnp.int32, 16)

        set0 = (fsrc0, frow0, fw0, rows0, gsem0, ssem0, dsem0)
        set1 = (fsrc1, frow1, fw1, rows1, gsem1, ssem1, dsem1)

        @pl.loop(0, RT // 16)
        def _(i):
            zero1_v[pl.ds(i * 16, 16)] = z16

        def wait_scatters(st):
            _, frow_x, fw_x, rows_x, _, ssem_x, dsem_x = st
            pltpu.make_async_copy(rows_x, a_sh.at[frow_x.at[0]], ssem_x).wait()
            pltpu.make_async_copy(fw_x, d_sh.at[frow_x.at[0]], dsem_x).wait()

        def complete_prev(st):
            fsrc_x, frow_x, fw_x, rows_x, gsem_x, ssem_x, dsem_x = st
            pltpu.make_async_copy(x_hbm.at[fsrc_x], rows_x, gsem_x).wait()

            @pl.loop(0, G // 16)
            def _(eg):
                wg = fw_x[pl.ds(eg * 16, 16)]
                for l in range(16):
                    e = eg * 16 + l
                    wb = lax.gather(
                        wg, jnp.full((16, 1), l, jnp.int32),
                        lax.GatherDimensionNumbers(
                            offset_dims=(), collapsed_slice_dims=(0,),
                            start_index_map=(0,)),
                        (1,), mode=lax.GatherScatterMode.PROMISE_IN_BOUNDS)
                    for cb in range(D // 16):
                        csl = pl.ds(cb * 16, 16)
                        rows_x[e, csl] = rows_x[e, csl] * wb

            pltpu.async_copy(rows_x, a_sh.at[frow_x.at[0]], ssem_x, add=True)
            pltpu.async_copy(fw_x, d_sh.at[frow_x.at[0]], dsem_x, add=True)

        def fire_step(cur, prev, nf):
            fsrc_c, frow_c, fw_c, rows_c, gsem_c, _, _ = cur

            @pl.when(nf >= 2)
            def _():
                wait_scatters(cur)

            for k in range(G // 16):
                sl = pl.ds(k * 16, 16)
                fsrc_c[sl] = stg_src[sl]
                frow_c[0, sl] = stg_row[sl]
                fw_c[sl] = stg_w[sl]
            for k in range(8):
                dsl = pl.ds(k * 16, 16)
                ssl = pl.ds(G + k * 16, 16)
                stg_src[dsl] = stg_src[ssl]
                stg_row[dsl] = stg_row[ssl]
                stg_w[dsl] = stg_w[ssl]
            pltpu.async_copy(x_hbm.at[fsrc_c], rows_c, gsem_c)

            @pl.when(nf >= 1)
            def _():
                complete_prev(prev)

        def force_fire(nf):
            @pl.when((nf & 1) == 0)
            def _():
                fire_step(set0, set1, nf)

            @pl.when((nf & 1) == 1)
            def _():
                fire_step(set1, set0, nf)

            return nf + 1

        def maybe_fire(c, nf):
            pred = c >= G

            @pl.when(pred & ((nf & 1) == 0))
            def _():
                fire_step(set0, set1, nf)

            @pl.when(pred & ((nf & 1) == 1))
            def _():
                fire_step(set1, set0, nf)

            return jnp.where(pred, c - G, c), jnp.where(pred, nf + 1, nf)

        @pl.loop(0, P)
        def _(p):
            lo = p * (2 * S) + scid * S
            hi = lo + S

            @pl.loop(0, G)
            def _(i):
                for k in range(D // 16):
                    rows0[i, pl.ds(k * 16, 16)] = z16

            for k in range(RT // G):
                pltpu.sync_copy(rows0,
                                a_sh.at[pl.ds(tid * RT + k * G, G)])
            pltpu.sync_copy(zero1_v, d_sh.at[pl.ds(tid * RT, RT)])
            plsc.subcore_barrier()

            def batch_body(b, carry):
                c, nf = carry
                for g in range(8):
                    off = b * 128 + g * 16
                    d16 = dst_v[pl.ds(off, 16)]
                    s16 = src_v[pl.ds(off, 16)]
                    w16 = w_v[pl.ds(off, 16)]
                    m = jnp.logical_and(d16 >= lo, d16 < hi)
                    plsc.store_compressed(stg_row.at[pl.ds(c, 16)],
                                          d16 - lo, mask=m)
                    plsc.store_compressed(stg_src.at[pl.ds(c, 16)],
                                          s16, mask=m)
                    plsc.store_compressed(stg_w.at[pl.ds(c, 16)],
                                          w16, mask=m)
                    cnt = plsc.all_reduce_population_count(m)
                    c = c + cnt[0]
                c, nf = maybe_fire(c, nf)
                c, nf = maybe_fire(c, nf)
                return (c, nf)

            def chunk_body(ch, carry):
                eb = tid * EPT + ch * LC
                cp0 = pltpu.async_copy(src_hbm.at[pl.ds(eb, LC)], src_v, lsem0)
                cp1 = pltpu.async_copy(dst_hbm.at[pl.ds(eb, LC)], dst_v, lsem1)
                cp2 = pltpu.async_copy(w_hbm.at[pl.ds(eb, LC)], w_v, lsem2)
                cp0.wait()
                cp1.wait()
                cp2.wait()
                return lax.fori_loop(0, LC // 128, batch_body, carry)

            c, nf = lax.fori_loop(0, NCH, chunk_body,
                                  (jnp.int32(0), jnp.int32(0)))

            for k in range(G // 16):
                sl = pl.ds(c + k * 16, 16)
                stg_row[sl] = iot + (k * 16)
                stg_src[sl] = zi16
                stg_w[sl] = z16

            def flush_body(carry):
                cc, nf = carry
                nf = force_fire(nf)
                return (cc - G, nf)

            c, nf = lax.while_loop(lambda cn: cn[0] > 0, flush_body, (c, nf))

            kf = nf

            @pl.when((kf >= 1) & (((kf - 1) & 1) == 0))
            def _():
                complete_prev(set0)

            @pl.when((kf >= 1) & (((kf - 1) & 1) == 1))
            def _():
                complete_prev(set1)

            @pl.when((kf >= 2) & ((kf & 1) == 0))
            def _():
                wait_scatters(set0)

            @pl.when((kf >= 2) & ((kf & 1) == 1))
            def _():
                wait_scatters(set1)

            @pl.when((kf >= 1) & (((kf - 1) & 1) == 0))
            def _():
                wait_scatters(set0)

            @pl.when((kf >= 1) & (((kf - 1) & 1) == 1))
            def _():
                wait_scatters(set1)

            plsc.subcore_barrier()

            gbase = p * (2 * S) + scid * S + tid * RT
            pltpu.sync_copy(a_sh.at[pl.ds(tid * RT, RT)],
                            a_hbm.at[pl.ds(gbase, RT)])
            pltpu.sync_copy(d_sh.at[pl.ds(tid * RT, RT)],
                            d_hbm.at[pl.ds(gbase, RT)])
            plsc.subcore_barrier()

    return sc_kernel(src_p, dst_p, w_p, x)


TN = 1000


def _tc_body(a_ref, d_ref, x_ref, wl_ref, ws_ref, bl_ref, bs_ref, o_ref):
    r = pl.program_id(1)
    dall = d_ref[...]
    colmask = lax.broadcasted_iota(jnp.int32, (TN, R), 1) == r
    dcol = jnp.sum(jnp.where(colmask, dall, 0.0), axis=1, keepdims=True)
    dinv = jnp.where(dcol > 0.0, 1.0 / dcol, 0.0)
    u = a_ref[...] * dinv
    acc = lax.dot_general(u, wl_ref[...], (((1,), (1,)), ((), ())),
                          preferred_element_type=jnp.float32)

    @pl.when(r == 0)
    def _():
        o_ref[...] = (
            lax.dot_general(x_ref[...], ws_ref[...],
                            (((1,), (1,)), ((), ())),
                            preferred_element_type=jnp.float32)
            + bl_ref[...] + bs_ref[...]
        )

    o_ref[...] += acc

    @pl.when(r == R - 1)
    def _():
        o_ref[...] = jnp.maximum(o_ref[...], 0.0)


def _tc_combine(a2, d2, x, W_lin, W_self, b_lin, b_self):
    return pl.pallas_call(
        _tc_body,
        out_shape=jax.ShapeDtypeStruct((N, D), jnp.float32),
        grid=(N // TN, R),
        in_specs=[
            pl.BlockSpec((TN, D), lambda i, r: (i, r)),
            pl.BlockSpec((TN, R), lambda i, r: (i, 0)),
            pl.BlockSpec((TN, D), lambda i, r: (i, 0)),
            pl.BlockSpec((D, D), lambda i, r: (0, r)),
            pl.BlockSpec((D, D), lambda i, r: (0, 0)),
            pl.BlockSpec((1, D), lambda i, r: (0, 0)),
            pl.BlockSpec((1, D), lambda i, r: (0, 0)),
        ],
        out_specs=pl.BlockSpec((TN, D), lambda i, r: (i, 0)),
    )(a2, d2, x, W_lin, W_self,
      b_lin.reshape(1, D), b_self.reshape(1, D))


def kernel(x, node_in, node_out, relation, edge_weight, W_lin, b_lin,
           W_self, b_self):
    pad = EP - E
    src_p = jnp.concatenate(
        [node_in.astype(jnp.int32), jnp.zeros((pad,), jnp.int32)])
    dst = node_out.astype(jnp.int32) * R + relation.astype(jnp.int32)
    dst_p = jnp.concatenate([dst, jnp.full((pad,), PAD_DST, jnp.int32)])
    w_p = jnp.concatenate([edge_weight, jnp.zeros((pad,), jnp.float32)])

    a, d = _sc_accumulate(src_p, dst_p, w_p, x)
    a2 = a.reshape(PTOT // R, R * D)
    d2 = d.reshape(PTOT // R, R)
    return _tc_combine(a2, d2, x, W_lin, W_self, b_lin, b_self)
